# baseline (speedup 1.0000x reference)
import functools

import jax
import jax.numpy as jnp
from jax import lax
from jax.experimental import pallas as pl
from jax.experimental.pallas import tpu as pltpu
from jax.experimental.pallas import tpu_sc as plsc

NUM_FIELDS = 26
VOCAB_PER_FIELD = 100000
EMBED_DIM = 128
BATCH = 4096

NC, NS, L = 2, 16, 16
NW = NC * NS
N_FLAT = BATCH * NUM_FIELDS
PER_W = N_FLAT // NW
CHUNK = 104
N_CHUNK = PER_W // CHUNK
NBUF = 4


@functools.partial(
    pl.kernel,
    out_type=jax.ShapeDtypeStruct((N_FLAT, EMBED_DIM), jnp.float32),
    mesh=plsc.VectorSubcoreMesh(core_axis_name="c", subcore_axis_name="s"),
    scratch_types=[
        pltpu.VMEM((PER_W,), jnp.int32),
    ] + [pltpu.VMEM((CHUNK, EMBED_DIM), jnp.float32) for _ in range(NBUF)]
      + [pltpu.SemaphoreType.DMA for _ in range(2 * NBUF)],
)
def _sok_gather(idx_hbm, table_hbm, out_hbm, idx_v, *rest):
    bufs = rest[:NBUF]
    gsem = rest[NBUF:2 * NBUF]
    ssem = rest[2 * NBUF:]
    wid = lax.axis_index("s") * NC + lax.axis_index("c")
    base = wid * PER_W

    pltpu.sync_copy(idx_hbm.at[pl.ds(base, PER_W)], idx_v)
    iota = lax.iota(jnp.int32, L)

    @pl.loop(0, PER_W // L, unroll=8)
    def _fuse(t):
        pos = base + t * L + iota
        off = lax.rem(pos, NUM_FIELDS) * VOCAB_PER_FIELD
        idx_v[pl.ds(t * L, L)] = idx_v[pl.ds(t * L, L)] + off

    def gather(j, b):
        pltpu.async_copy(table_hbm.at[idx_v.at[pl.ds(j * CHUNK, CHUNK)]],
                         bufs[b], gsem[b])

    def wait_gather(b):
        pltpu.make_async_copy(out_hbm.at[pl.ds(0, CHUNK)], bufs[b], gsem[b]).wait()

    def store(j, b):
        pltpu.async_copy(bufs[b], out_hbm.at[pl.ds((base + j * CHUNK), CHUNK)],
                         ssem[b])

    def wait_store(b):
        pltpu.make_async_copy(bufs[b], out_hbm.at[pl.ds(0, CHUNK)], ssem[b]).wait()

    for b in range(NBUF):
        gather(b, b)

    @pl.loop(0, N_CHUNK - NBUF, step=NBUF)
    def _main(j0):
        for b in range(NBUF):
            j = j0 + b
            wait_gather(b)
            store(j, b)
            wait_store(b)
            gather(j + NBUF, b)

    for b in range(NBUF):
        wait_gather(b)
        store(N_CHUNK - NBUF + b, b)
        wait_store(b)


RT_S = 16


def _retile_body(x_ref, y_ref):
    for s in range(RT_S):
        y_ref[s] = x_ref[pl.ds(s * NUM_FIELDS, NUM_FIELDS), :]


_retile = pl.pallas_call(
    _retile_body,
    grid=(BATCH // RT_S,),
    in_specs=[pl.BlockSpec((RT_S * NUM_FIELDS, EMBED_DIM), lambda i: (i, 0))],
    out_specs=pl.BlockSpec((RT_S, NUM_FIELDS, EMBED_DIM), lambda i: (i, 0, 0)),
    out_shape=jax.ShapeDtypeStruct((BATCH, NUM_FIELDS, EMBED_DIM), jnp.float32),
)


def kernel(inputs, table):
    flat_ids = inputs.reshape(-1)
    out = _sok_gather(flat_ids, table)
    return _retile(out)


# speedup vs baseline: 2.4384x; 2.4384x over previous
import functools

import jax
import jax.numpy as jnp
from jax import lax
from jax.experimental import pallas as pl
from jax.experimental.pallas import tpu as pltpu
from jax.experimental.pallas import tpu_sc as plsc

NUM_FIELDS = 26
VOCAB_PER_FIELD = 100000
EMBED_DIM = 128
BATCH = 4096

NC, NS, L = 2, 16, 16
NW = NC * NS
N_FLAT = BATCH * NUM_FIELDS
PER_W = N_FLAT // NW
SAMP_W = BATCH // NW
SPC = 4
CHUNK = SPC * NUM_FIELDS
N_CHUNK = PER_W // CHUNK
NBUF = 4


@functools.partial(
    pl.kernel,
    out_type=jax.ShapeDtypeStruct((BATCH, NUM_FIELDS, EMBED_DIM), jnp.float32),
    mesh=plsc.VectorSubcoreMesh(core_axis_name="c", subcore_axis_name="s"),
    scratch_types=[
        pltpu.VMEM((PER_W,), jnp.int32),
    ] + [pltpu.VMEM((CHUNK, EMBED_DIM), jnp.float32) for _ in range(NBUF)]
      + [pltpu.SemaphoreType.DMA for _ in range(2 * NBUF)],
)
def _sok_gather(idx_hbm, table_hbm, out_hbm, idx_v, *rest):
    bufs = rest[:NBUF]
    gsem = rest[NBUF:2 * NBUF]
    ssem = rest[2 * NBUF:]
    wid = lax.axis_index("s") * NC + lax.axis_index("c")
    base = wid * PER_W
    samp0 = wid * SAMP_W

    pltpu.sync_copy(idx_hbm.at[pl.ds(base, PER_W)], idx_v)
    iota = lax.iota(jnp.int32, L)

    @pl.loop(0, PER_W // L, unroll=8)
    def _fuse(t):
        pos = base + t * L + iota
        off = lax.rem(pos, NUM_FIELDS) * VOCAB_PER_FIELD
        idx_v[pl.ds(t * L, L)] = idx_v[pl.ds(t * L, L)] + off

    def gather(j, b):
        pltpu.async_copy(table_hbm.at[idx_v.at[pl.ds(j * CHUNK, CHUNK)]],
                         bufs[b], gsem[b])

    def wait_gather(b):
        pltpu.make_async_copy(table_hbm.at[pl.ds(0, CHUNK)], bufs[b],
                              gsem[b]).wait()

    def store(j, b):
        for s in range(SPC):
            pltpu.async_copy(bufs[b].at[pl.ds(s * NUM_FIELDS, NUM_FIELDS)],
                             out_hbm.at[samp0 + j * SPC + s], ssem[b])

    def wait_store(b):
        for s in range(SPC):
            pltpu.make_async_copy(bufs[b].at[pl.ds(s * NUM_FIELDS, NUM_FIELDS)],
                                  out_hbm.at[0], ssem[b]).wait()

    for b in range(NBUF):
        gather(b, b)

    @pl.loop(0, N_CHUNK - NBUF, step=NBUF)
    def _main(j0):
        for b in range(NBUF):
            j = j0 + b
            wait_gather(b)
            store(j, b)
            wait_store(b)
            gather(j + NBUF, b)

    for b in range(NBUF):
        wait_gather(b)
        store(N_CHUNK - NBUF + b, b)
        wait_store(b)


def kernel(inputs, table):
    flat_ids = inputs.reshape(-1)
    return _sok_gather(flat_ids, table)


# speedup vs baseline: 4.5194x; 1.8534x over previous
import functools

import jax
import jax.numpy as jnp
from jax import lax
from jax.experimental import pallas as pl
from jax.experimental.pallas import tpu as pltpu
from jax.experimental.pallas import tpu_sc as plsc

NUM_FIELDS = 26
VOCAB_PER_FIELD = 100000
EMBED_DIM = 128
BATCH = 4096

NC, NS, L = 2, 16, 16
NW = NC * NS
N_FLAT = BATCH * NUM_FIELDS
PER_W = N_FLAT // NW
CHUNK = 104
N_CHUNK = PER_W // CHUNK
NBUF = 4


@functools.partial(
    pl.kernel,
    out_type=jax.ShapeDtypeStruct((N_FLAT, EMBED_DIM), jnp.float32),
    mesh=plsc.VectorSubcoreMesh(core_axis_name="c", subcore_axis_name="s"),
    scratch_types=[
        pltpu.VMEM((PER_W,), jnp.int32),
    ] + [pltpu.VMEM((CHUNK, EMBED_DIM), jnp.float32) for _ in range(NBUF)]
      + [pltpu.SemaphoreType.DMA for _ in range(2 * NBUF)],
)
def _sok_gather(idx_hbm, table_hbm, out_hbm, idx_v, *rest):
    bufs = rest[:NBUF]
    gsem = rest[NBUF:2 * NBUF]
    ssem = rest[2 * NBUF:]
    wid = lax.axis_index("s") * NC + lax.axis_index("c")
    base = wid * PER_W

    pltpu.sync_copy(idx_hbm.at[pl.ds(base, PER_W)], idx_v)
    iota = lax.iota(jnp.int32, L)

    @pl.loop(0, PER_W // L, unroll=8)
    def _fuse(t):
        pos = base + t * L + iota
        off = lax.div(pos, BATCH) * VOCAB_PER_FIELD
        idx_v[pl.ds(t * L, L)] = idx_v[pl.ds(t * L, L)] + off

    def gather(j, b):
        pltpu.async_copy(table_hbm.at[idx_v.at[pl.ds(j * CHUNK, CHUNK)]],
                         bufs[b], gsem[b])

    def wait_gather(b):
        pltpu.make_async_copy(table_hbm.at[pl.ds(0, CHUNK)], bufs[b],
                              gsem[b]).wait()

    def store(j, b):
        pltpu.async_copy(bufs[b], out_hbm.at[pl.ds(base + j * CHUNK, CHUNK)],
                         ssem[b])

    def wait_store(b):
        pltpu.make_async_copy(bufs[b], out_hbm.at[pl.ds(0, CHUNK)],
                              ssem[b]).wait()

    for b in range(NBUF):
        gather(b, b)

    @pl.loop(0, N_CHUNK - NBUF, step=NBUF)
    def _main(j0):
        for b in range(NBUF):
            j = j0 + b
            wait_gather(b)
            store(j, b)
            wait_store(b)
            gather(j + NBUF, b)

    for b in range(NBUF):
        wait_gather(b)
        store(N_CHUNK - NBUF + b, b)
        wait_store(b)


def kernel(inputs, table):
    flat_ids = inputs.T.reshape(-1)
    out = _sok_gather(flat_ids, table)
    return out.reshape(NUM_FIELDS, BATCH, EMBED_DIM).transpose(1, 0, 2)


# speedup vs baseline: 4.5198x; 1.0001x over previous
import functools

import jax
import jax.numpy as jnp
from jax import lax
from jax.experimental import pallas as pl
from jax.experimental.pallas import tpu as pltpu
from jax.experimental.pallas import tpu_sc as plsc

NUM_FIELDS = 26
VOCAB_PER_FIELD = 100000
EMBED_DIM = 128
BATCH = 4096

NC, NS, L = 2, 16, 16
NW = NC * NS
N_FLAT = BATCH * NUM_FIELDS
PER_W = N_FLAT // NW
CHUNK = 104
N_CHUNK = PER_W // CHUNK
NBUF = 8
AHEAD = 4


@functools.partial(
    pl.kernel,
    out_type=jax.ShapeDtypeStruct((N_FLAT, EMBED_DIM), jnp.float32),
    mesh=plsc.VectorSubcoreMesh(core_axis_name="c", subcore_axis_name="s"),
    scratch_types=[
        pltpu.VMEM((PER_W,), jnp.int32),
    ] + [pltpu.VMEM((CHUNK, EMBED_DIM), jnp.float32) for _ in range(NBUF)]
      + [pltpu.SemaphoreType.DMA for _ in range(2 * NBUF)],
)
def _sok_gather(idx_hbm, table_hbm, out_hbm, idx_v, *rest):
    bufs = rest[:NBUF]
    gsem = rest[NBUF:2 * NBUF]
    ssem = rest[2 * NBUF:]
    wid = lax.axis_index("s") * NC + lax.axis_index("c")
    base = wid * PER_W

    pltpu.sync_copy(idx_hbm.at[pl.ds(base, PER_W)], idx_v)
    iota = lax.iota(jnp.int32, L)

    @pl.loop(0, PER_W // L, unroll=8)
    def _fuse(t):
        pos = base + t * L + iota
        off = lax.div(pos, BATCH) * VOCAB_PER_FIELD
        idx_v[pl.ds(t * L, L)] = idx_v[pl.ds(t * L, L)] + off

    def gather(j, b):
        pltpu.async_copy(table_hbm.at[idx_v.at[pl.ds(j * CHUNK, CHUNK)]],
                         bufs[b], gsem[b])

    def wait_gather(b):
        pltpu.make_async_copy(table_hbm.at[pl.ds(0, CHUNK)], bufs[b],
                              gsem[b]).wait()

    def store(j, b):
        pltpu.async_copy(bufs[b], out_hbm.at[pl.ds(base + j * CHUNK, CHUNK)],
                         ssem[b])

    def wait_store(b):
        pltpu.make_async_copy(bufs[b], out_hbm.at[pl.ds(0, CHUNK)],
                              ssem[b]).wait()

    def step(j, b, prefetch, wait_prev):
        wait_gather(b)
        store(j, b)
        if prefetch:
            bn = (b + AHEAD) % NBUF
            if wait_prev:
                wait_store(bn)
            gather(j + AHEAD, bn)

    for b in range(AHEAD):
        gather(b, b)
    for j in range(NBUF):
        step(j, j, True, j + AHEAD >= NBUF)

    @pl.loop(NBUF, N_CHUNK - NBUF, step=NBUF)
    def _main(j0):
        for bb in range(NBUF):
            step(j0 + bb, bb, True, True)

    for j in range(N_CHUNK - NBUF, N_CHUNK):
        step(j, j % NBUF, j + AHEAD < N_CHUNK, True)
    for j in range(N_CHUNK - NBUF, N_CHUNK):
        wait_store(j % NBUF)


def kernel(inputs, table):
    flat_ids = inputs.T.reshape(-1)
    out = _sok_gather(flat_ids, table)
    return out.reshape(NUM_FIELDS, BATCH, EMBED_DIM).transpose(1, 0, 2)


# speedup vs baseline: 4.5700x; 1.0111x over previous
import functools

import jax
import jax.numpy as jnp
from jax import lax
from jax.experimental import pallas as pl
from jax.experimental.pallas import tpu as pltpu
from jax.experimental.pallas import tpu_sc as plsc

NUM_FIELDS = 26
VOCAB_PER_FIELD = 100000
EMBED_DIM = 128
BATCH = 4096

NC, NS, L = 2, 16, 16
NW = NC * NS
N_FLAT = BATCH * NUM_FIELDS
PER_W = N_FLAT // NW
CHUNK = 104
N_CHUNK = PER_W // CHUNK
NBUF = 8
AHEAD = 6


@functools.partial(
    pl.kernel,
    out_type=jax.ShapeDtypeStruct((N_FLAT, EMBED_DIM), jnp.float32),
    mesh=plsc.VectorSubcoreMesh(core_axis_name="c", subcore_axis_name="s"),
    scratch_types=[
        pltpu.VMEM((PER_W,), jnp.int32),
    ] + [pltpu.VMEM((CHUNK, EMBED_DIM), jnp.float32) for _ in range(NBUF)]
      + [pltpu.SemaphoreType.DMA for _ in range(2 * NBUF)],
)
def _sok_gather(idx_hbm, table_hbm, out_hbm, idx_v, *rest):
    bufs = rest[:NBUF]
    gsem = rest[NBUF:2 * NBUF]
    ssem = rest[2 * NBUF:]
    wid = lax.axis_index("s") * NC + lax.axis_index("c")
    base = wid * PER_W

    pltpu.sync_copy(idx_hbm.at[pl.ds(base, PER_W)], idx_v)
    iota = lax.iota(jnp.int32, L)

    @pl.loop(0, PER_W // L, unroll=8)
    def _fuse(t):
        pos = base + t * L + iota
        off = lax.div(pos, BATCH) * VOCAB_PER_FIELD
        idx_v[pl.ds(t * L, L)] = idx_v[pl.ds(t * L, L)] + off

    def gather(j, b):
        pltpu.async_copy(table_hbm.at[idx_v.at[pl.ds(j * CHUNK, CHUNK)]],
                         bufs[b], gsem[b])

    def wait_gather(b):
        pltpu.make_async_copy(table_hbm.at[pl.ds(0, CHUNK)], bufs[b],
                              gsem[b]).wait()

    def store(j, b):
        pltpu.async_copy(bufs[b], out_hbm.at[pl.ds(base + j * CHUNK, CHUNK)],
                         ssem[b])

    def wait_store(b):
        pltpu.make_async_copy(bufs[b], out_hbm.at[pl.ds(0, CHUNK)],
                              ssem[b]).wait()

    def step(j, b, prefetch, wait_prev):
        wait_gather(b)
        store(j, b)
        if prefetch:
            bn = (b + AHEAD) % NBUF
            if wait_prev:
                wait_store(bn)
            gather(j + AHEAD, bn)

    for b in range(AHEAD):
        gather(b, b)
    for j in range(NBUF):
        step(j, j, True, j + AHEAD >= NBUF)

    @pl.loop(NBUF, N_CHUNK - NBUF, step=NBUF)
    def _main(j0):
        for bb in range(NBUF):
            step(j0 + bb, bb, True, True)

    for j in range(N_CHUNK - NBUF, N_CHUNK):
        step(j, j % NBUF, j + AHEAD < N_CHUNK, True)
    for j in range(N_CHUNK - NBUF, N_CHUNK):
        wait_store(j % NBUF)


def kernel(inputs, table):
    flat_ids = inputs.T.reshape(-1)
    out = _sok_gather(flat_ids, table)
    return out.reshape(NUM_FIELDS, BATCH, EMBED_DIM).transpose(1, 0, 2)
